# Initial kernel scaffold; baseline (speedup 1.0000x reference)
import jax
import jax.numpy as jnp
from jax.experimental import pallas as pl


def kernel(x, edge_index, edge_weights, W, b):
    raise NotImplementedError("write your pallas kernel here")


# speedup vs baseline: 14.5611x; 14.5611x over previous
import dataclasses
import functools

import jax
import jax.numpy as jnp
from jax import lax
from jax.experimental import pallas as pl
from jax.experimental.pallas import tpu as pltpu
from jax.experimental.pallas import tpu_sc as plsc

_NC = 2
_NS = 16
_L = 16
_WIN = 128


def _sc_params():
    cp = pltpu.CompilerParams()
    if "needs_layout_passes" in pltpu.CompilerParams.__dataclass_fields__:
        cp = dataclasses.replace(cp, needs_layout_passes=False)
    return cp


def _sc_aggregate(x, edge_index, w, n_pad):
    N, D = x.shape
    E = edge_index.shape[1]
    NP = n_pad
    WN = E // _WIN
    assert E % _WIN == 0 and D % _L == 0 and NP % (_NS * _WIN) == 0
    RPS = NP // _NS
    NW = _NC * _NS

    mesh = plsc.VectorSubcoreMesh(core_axis_name="c", subcore_axis_name="s")

    @functools.partial(
        pl.kernel,
        out_type=(
            jax.ShapeDtypeStruct((_NC, NP, D), jnp.float32),
            jax.ShapeDtypeStruct((NP,), jnp.float32),
        ),
        mesh=mesh,
        scratch_types=[
            pltpu.VMEM_SHARED((NP,), jnp.float32),
            pltpu.VMEM_SHARED((NP, D), jnp.float32),
            pltpu.VMEM((NP,), jnp.float32),
            pltpu.VMEM((NP,), jnp.float32),
            pltpu.VMEM((_WIN, D), jnp.float32),
            pltpu.VMEM((RPS,), jnp.float32),
            pltpu.VMEM((_WIN,), jnp.int32),
            pltpu.VMEM((1, _WIN), jnp.int32),
            pltpu.VMEM((_WIN,), jnp.float32),
            pltpu.VMEM((_WIN,), jnp.float32),
        ],
        compiler_params=_sc_params(),
    )
    def k(x_hbm, ei_hbm, w_hbm, p_hbm, deg_hbm,
          deg_sh, agg_sh, deg_l, isd_l, rows, zbuf, sidx, didx, wbuf, cbuf):
        c = lax.axis_index("c")
        s = lax.axis_index("s")
        wid = s * _NC + c
        zero16 = jnp.zeros((_L,), jnp.float32)
        src_hbm = ei_hbm.at[0]
        dst_hbm = ei_hbm.at[1]

        @pl.loop(0, _WIN)
        def _(r):
            for j in range(D // _L):
                rows[r, pl.ds(j * _L, _L)] = zero16

        @pl.loop(0, RPS // _L)
        def _(t):
            zbuf[pl.ds(t * _L, _L)] = zero16

        for t in range(RPS // _WIN):
            pltpu.sync_copy(rows, agg_sh.at[pl.ds(s * RPS + t * _WIN, _WIN), :])
        pltpu.sync_copy(zbuf, deg_sh.at[pl.ds(s * RPS, RPS)])
        plsc.subcore_barrier()

        nw1 = WN // _NS + jnp.where(s < (WN % _NS), 1, 0)

        @pl.loop(0, nw1)
        def _(k1):
            base = (s + k1 * _NS) * _WIN
            pltpu.sync_copy(dst_hbm.at[pl.ds(base, _WIN)], didx.at[0])
            pltpu.sync_copy(w_hbm.at[pl.ds(base, _WIN)], wbuf)
            pltpu.sync_copy(wbuf, deg_sh.at[didx.at[0]], add=True)

        plsc.subcore_barrier()

        pltpu.sync_copy(deg_sh, deg_l)

        @pl.loop(0, NP // _L)
        def _(t):
            d = deg_l[pl.ds(t * _L, _L)] + 1.0
            i = plsc.bitcast(d, jnp.int32)
            y = plsc.bitcast(jnp.int32(0x5F3759DF) - (i >> 1), jnp.float32)
            y = y * (1.5 - 0.5 * d * y * y)
            y = y * (1.5 - 0.5 * d * y * y)
            y = y * (1.5 - 0.5 * d * y * y)
            isd_l[pl.ds(t * _L, _L)] = y

        @pl.when(c == 0)
        def _():
            pltpu.sync_copy(deg_sh.at[pl.ds(s * RPS, RPS)],
                            deg_hbm.at[pl.ds(s * RPS, RPS)])

        nw3 = WN // NW + jnp.where(wid < (WN % NW), 1, 0)

        @pl.loop(0, nw3)
        def _(k3):
            base = (wid + k3 * NW) * _WIN
            pltpu.sync_copy(src_hbm.at[pl.ds(base, _WIN)], sidx)
            pltpu.sync_copy(dst_hbm.at[pl.ds(base, _WIN)], didx.at[0])
            pltpu.sync_copy(w_hbm.at[pl.ds(base, _WIN)], wbuf)
            pltpu.sync_copy(x_hbm.at[sidx], rows)
            for kk in range(_WIN // _L):
                s16 = sidx[pl.ds(kk * _L, _L)]
                isd_s = plsc.load_gather(isd_l, [s16])
                cbuf[pl.ds(kk * _L, _L)] = wbuf[pl.ds(kk * _L, _L)] * isd_s

            @pl.loop(0, _WIN // _L)
            def _(g):
                c16 = cbuf[pl.ds(g * _L, _L)]
                for l in range(_L):
                    ce = c16[l]
                    e = g * _L + l
                    for j in range(D // _L):
                        rows[e, pl.ds(j * _L, _L)] = rows[e, pl.ds(j * _L, _L)] * ce

            pltpu.sync_copy(rows, agg_sh.at[didx.at[0]], add=True)

        plsc.subcore_barrier()

        for t in range(RPS // _WIN):
            sl = pl.ds(s * RPS + t * _WIN, _WIN)
            pltpu.sync_copy(agg_sh.at[sl, :], p_hbm.at[c].at[sl, :])

    return k(x, edge_index, w)


def _tc_finish(P, x, deg2, W, b2):
    N, D = x.shape
    RB = 2000
    assert N % RB == 0

    def body(p0_r, p1_r, x_r, deg_r, w_r, b_r, o_r):
        deg = deg_r[...] + 1.0
        agg = lax.rsqrt(deg) * (p0_r[0] + p1_r[0]) + x_r[...] / deg
        y = jnp.dot(agg, w_r[...], preferred_element_type=jnp.float32) + b_r[...]
        o_r[...] = jnp.maximum(y, 0.0)

    return pl.pallas_call(
        body,
        grid=(N // RB,),
        in_specs=[
            pl.BlockSpec((1, RB, D), lambda i: (0, i, 0)),
            pl.BlockSpec((1, RB, D), lambda i: (1, i, 0)),
            pl.BlockSpec((RB, D), lambda i: (i, 0)),
            pl.BlockSpec((RB, 1), lambda i: (i, 0)),
            pl.BlockSpec((D, D), lambda i: (0, 0)),
            pl.BlockSpec((1, D), lambda i: (0, 0)),
        ],
        out_specs=pl.BlockSpec((RB, D), lambda i: (i, 0)),
        out_shape=jax.ShapeDtypeStruct((N, D), jnp.float32),
    )(P, P, x, deg2, W, b2)


def kernel(x, edge_index, edge_weights, W, b):
    N, D = x.shape
    NP = 10240
    P, deg_raw = _sc_aggregate(x, edge_index, edge_weights, NP)
    deg2 = deg_raw[:N].reshape(N, 1)
    b2 = b.reshape(1, D)
    return _tc_finish(P, x, deg2, W, b2)


# speedup vs baseline: 29.3654x; 2.0167x over previous
import dataclasses
import functools

import jax
import jax.numpy as jnp
from jax import lax
from jax.experimental import pallas as pl
from jax.experimental.pallas import tpu as pltpu
from jax.experimental.pallas import tpu_sc as plsc

_NC = 2
_NS = 16
_L = 16
_WIN = 128
_CH = 8


def _sc_params():
    cp = pltpu.CompilerParams()
    if "needs_layout_passes" in pltpu.CompilerParams.__dataclass_fields__:
        cp = dataclasses.replace(cp, needs_layout_passes=False)
    return cp


def _sc_aggregate(x, src2, dst2, w2, n_pad):
    N, D = x.shape
    WN = src2.shape[0]
    NP = n_pad
    RPS = NP // _NS
    NW = _NC * _NS
    DG = D // _L

    W3 = WN // NW
    W1 = WN // _NS
    assert WN % NW == 0 and W3 % _CH == 0 and W1 % _CH == 0

    mesh = plsc.VectorSubcoreMesh(core_axis_name="c", subcore_axis_name="s")

    @functools.partial(
        pl.kernel,
        out_type=(
            jax.ShapeDtypeStruct((_NC, NP, D), jnp.float32),
            jax.ShapeDtypeStruct((NP,), jnp.float32),
        ),
        mesh=mesh,
        scratch_types=[
            pltpu.VMEM_SHARED((NP,), jnp.float32),
            pltpu.VMEM_SHARED((NP, D), jnp.float32),
            pltpu.VMEM((NP,), jnp.float32),
            pltpu.VMEM((2, _WIN, D), jnp.float32),
            pltpu.VMEM((_WIN,), jnp.float32),
            pltpu.VMEM((_CH, _WIN), jnp.int32),
            pltpu.VMEM((_CH, _WIN), jnp.int32),
            pltpu.VMEM((_CH, _WIN), jnp.float32),
            pltpu.VMEM((_WIN,), jnp.float32),
            pltpu.SemaphoreType.DMA,
            pltpu.SemaphoreType.DMA,
            pltpu.SemaphoreType.DMA,
            pltpu.SemaphoreType.DMA,
            pltpu.SemaphoreType.DMA,
        ],
        compiler_params=_sc_params(),
    )
    def k(x_hbm, s2_hbm, d2_hbm, w2_hbm, p_hbm, deg_hbm,
          deg_sh, agg_sh, isd_l, rows2, zbuf,
          s_chunk, d_chunk, w_chunk, cbuf,
          gsem0, gsem1, tsem0, tsem1, psem):
        c = lax.axis_index("c")
        s = lax.axis_index("s")
        wid = s * _NC + c
        zero16 = jnp.zeros((_L,), jnp.float32)
        gsem = (gsem0, gsem1)
        tsem = (tsem0, tsem1)

        @pl.loop(0, _WIN)
        def _(r):
            for j in range(DG):
                rows2[0, r, pl.ds(j * _L, _L)] = zero16

        @pl.loop(0, _WIN // _L)
        def _(t):
            zbuf[pl.ds(t * _L, _L)] = zero16

        for t in range(RPS // _WIN):
            pltpu.sync_copy(rows2.at[0],
                            agg_sh.at[pl.ds(s * RPS + t * _WIN, _WIN), :])
            pltpu.sync_copy(zbuf, deg_sh.at[pl.ds(s * RPS + t * _WIN, _WIN)])
        plsc.subcore_barrier()

        start1 = s * W1

        @pl.loop(0, W1 // _CH)
        def _(cki):
            w0 = start1 + cki * _CH
            pltpu.sync_copy(d2_hbm.at[pl.ds(w0, _CH), :], d_chunk)
            pltpu.sync_copy(w2_hbm.at[pl.ds(w0, _CH), :], w_chunk)
            descs = [
                pltpu.async_copy(w_chunk.at[j], deg_sh.at[d_chunk.at[j]],
                                 psem, add=True)
                for j in range(_CH)
            ]
            for d in descs:
                d.wait()

        plsc.subcore_barrier()

        pltpu.sync_copy(deg_sh, isd_l)

        @pl.loop(0, NP // _L)
        def _(t):
            d = isd_l[pl.ds(t * _L, _L)] + 1.0
            i = plsc.bitcast(d, jnp.int32)
            y = plsc.bitcast(jnp.int32(0x5F3759DF) - (i >> 1), jnp.float32)
            y = y * (1.5 - 0.5 * d * y * y)
            y = y * (1.5 - 0.5 * d * y * y)
            y = y * (1.5 - 0.5 * d * y * y)
            isd_l[pl.ds(t * _L, _L)] = y

        @pl.when(c == 0)
        def _():
            pltpu.sync_copy(deg_sh.at[pl.ds(s * RPS, RPS)],
                            deg_hbm.at[pl.ds(s * RPS, RPS)])

        start3 = wid * W3

        def scale_window(j, b):
            for kk in range(_WIN // _L):
                s16 = s_chunk[j, pl.ds(kk * _L, _L)]
                isd_s = plsc.load_gather(isd_l, [s16])
                cbuf[pl.ds(kk * _L, _L)] = w_chunk[j, pl.ds(kk * _L, _L)] * isd_s

            @pl.loop(0, _WIN // _L)
            def _(g):
                c16 = cbuf[pl.ds(g * _L, _L)]
                for l in range(_L):
                    ce = c16[l]
                    e = g * _L + l
                    for jj in range(DG):
                        rows2[b, e, pl.ds(jj * _L, _L)] = (
                            rows2[b, e, pl.ds(jj * _L, _L)] * ce)

        @pl.loop(0, W3 // _CH)
        def _(cki):
            w0 = start3 + cki * _CH
            pltpu.sync_copy(s2_hbm.at[pl.ds(w0, _CH), :], s_chunk)
            pltpu.sync_copy(d2_hbm.at[pl.ds(w0, _CH), :], d_chunk)
            pltpu.sync_copy(w2_hbm.at[pl.ds(w0, _CH), :], w_chunk)
            g = [None, None]
            t = [None, None]
            g[0] = pltpu.async_copy(x_hbm.at[s_chunk.at[0]], rows2.at[0], gsem[0])
            for j in range(_CH):
                b = j % 2
                nb = (j + 1) % 2
                if j + 1 < _CH:
                    if t[nb] is not None:
                        t[nb].wait()
                    g[nb] = pltpu.async_copy(x_hbm.at[s_chunk.at[j + 1]],
                                             rows2.at[nb], gsem[nb])
                g[b].wait()
                scale_window(j, b)
                t[b] = pltpu.async_copy(rows2.at[b], agg_sh.at[d_chunk.at[j]],
                                        tsem[b], add=True)
            t[0].wait()
            t[1].wait()

        plsc.subcore_barrier()

        for t in range(RPS // _WIN):
            sl = pl.ds(s * RPS + t * _WIN, _WIN)
            pltpu.sync_copy(agg_sh.at[sl, :], p_hbm.at[c].at[sl, :])

    return k(x, src2, dst2, w2)


def _tc_finish(P, x, deg2, W, b2):
    N, D = x.shape
    RB = 2000
    assert N % RB == 0

    def body(p0_r, p1_r, x_r, deg_r, w_r, b_r, o_r):
        deg = deg_r[...] + 1.0
        agg = lax.rsqrt(deg) * (p0_r[0] + p1_r[0]) + x_r[...] / deg
        y = jnp.dot(agg, w_r[...], preferred_element_type=jnp.float32) + b_r[...]
        o_r[...] = jnp.maximum(y, 0.0)

    return pl.pallas_call(
        body,
        grid=(N // RB,),
        in_specs=[
            pl.BlockSpec((1, RB, D), lambda i: (0, i, 0)),
            pl.BlockSpec((1, RB, D), lambda i: (1, i, 0)),
            pl.BlockSpec((RB, D), lambda i: (i, 0)),
            pl.BlockSpec((RB, 1), lambda i: (i, 0)),
            pl.BlockSpec((D, D), lambda i: (0, 0)),
            pl.BlockSpec((1, D), lambda i: (0, 0)),
        ],
        out_specs=pl.BlockSpec((RB, D), lambda i: (i, 0)),
        out_shape=jax.ShapeDtypeStruct((N, D), jnp.float32),
    )(P, P, x, deg2, W, b2)


def kernel(x, edge_index, edge_weights, W, b):
    N, D = x.shape
    E = edge_index.shape[1]
    NP = 10240
    unit = _WIN * _NC * _NS * _CH
    EP = -(-E // unit) * unit
    pad = EP - E
    pad_idx = jnp.arange(pad, dtype=jnp.int32) % jnp.int32(N)
    src2 = jnp.concatenate([edge_index[0], pad_idx]).reshape(EP // _WIN, _WIN)
    dst2 = jnp.concatenate([edge_index[1], pad_idx]).reshape(EP // _WIN, _WIN)
    w2 = jnp.concatenate(
        [edge_weights, jnp.zeros((pad,), jnp.float32)]).reshape(EP // _WIN, _WIN)
    P, deg_raw = _sc_aggregate(x, src2, dst2, w2, NP)
    deg2 = deg_raw[:N].reshape(N, 1)
    b2 = b.reshape(1, D)
    return _tc_finish(P, x, deg2, W, b2)


# speedup vs baseline: 34.7090x; 1.1820x over previous
import dataclasses
import functools

import jax
import jax.numpy as jnp
from jax import lax
from jax.experimental import pallas as pl
from jax.experimental.pallas import tpu as pltpu
from jax.experimental.pallas import tpu_sc as plsc

_NC = 2
_NS = 16
_L = 16
_WIN = 128
_CH = 4


def _sc_params():
    cp = pltpu.CompilerParams()
    if "needs_layout_passes" in pltpu.CompilerParams.__dataclass_fields__:
        cp = dataclasses.replace(cp, needs_layout_passes=False)
    return cp


def _sc_aggregate(x, src2, dst2, w2, n_pad):
    N, D = x.shape
    WN = src2.shape[0]
    NP = n_pad
    RPS = NP // _NS
    NW = _NC * _NS
    DG = D // _L

    W3 = WN // NW
    W1 = WN // _NS
    NCH3 = W3 // _CH
    NCH1 = W1 // _CH
    assert WN % NW == 0 and W3 % _CH == 0 and W1 % _CH == 0

    mesh = plsc.VectorSubcoreMesh(core_axis_name="c", subcore_axis_name="s")

    @functools.partial(
        pl.kernel,
        out_type=(
            jax.ShapeDtypeStruct((_NC, NP, D), jnp.float32),
            jax.ShapeDtypeStruct((NP,), jnp.float32),
        ),
        mesh=mesh,
        scratch_types=[
            pltpu.VMEM_SHARED((NP,), jnp.float32),
            pltpu.VMEM_SHARED((NP, D), jnp.float32),
            pltpu.VMEM((NP,), jnp.float32),
            pltpu.VMEM((2, _WIN, D), jnp.float32),
            pltpu.VMEM((_WIN,), jnp.float32),
            pltpu.VMEM((2, _CH, _WIN), jnp.int32),
            pltpu.VMEM((2, _CH, _WIN), jnp.int32),
            pltpu.VMEM((2, _CH, _WIN), jnp.float32),
            pltpu.VMEM((_WIN,), jnp.float32),
            pltpu.SemaphoreType.DMA,
            pltpu.SemaphoreType.DMA,
            pltpu.SemaphoreType.DMA,
            pltpu.SemaphoreType.DMA,
            pltpu.SemaphoreType.DMA,
            pltpu.SemaphoreType.DMA,
        ],
        compiler_params=_sc_params(),
    )
    def k(x_hbm, s2_hbm, d2_hbm, w2_hbm, p_hbm, deg_hbm,
          deg_sh, agg_sh, isd_l, rows2, zbuf,
          s_chunk2, d_chunk2, w_chunk2, cbuf,
          gsem0, gsem1, tsem0, tsem1, stsem, psem):
        c = lax.axis_index("c")
        s = lax.axis_index("s")
        wid = s * _NC + c
        zero16 = jnp.zeros((_L,), jnp.float32)
        gsem = (gsem0, gsem1)
        tsem = (tsem0, tsem1)

        @pl.loop(0, _WIN)
        def _(r):
            for j in range(DG):
                rows2[0, r, pl.ds(j * _L, _L)] = zero16

        @pl.loop(0, _WIN // _L)
        def _(t):
            zbuf[pl.ds(t * _L, _L)] = zero16

        for t in range(RPS // _WIN):
            pltpu.sync_copy(rows2.at[0],
                            agg_sh.at[pl.ds(s * RPS + t * _WIN, _WIN), :])
            pltpu.sync_copy(zbuf, deg_sh.at[pl.ds(s * RPS + t * _WIN, _WIN)])
        plsc.subcore_barrier()

        start1 = s * W1
        pltpu.sync_copy(d2_hbm.at[pl.ds(start1, _CH), :], d_chunk2.at[0])
        pltpu.sync_copy(w2_hbm.at[pl.ds(start1, _CH), :], w_chunk2.at[0])

        @pl.loop(0, NCH1)
        def _(cki):
            cs = cki % 2
            ns = (cki + 1) % 2
            w0n = start1 + (cki + 1) * _CH

            @pl.when(cki < NCH1 - 1)
            def _():
                pltpu.async_copy(d2_hbm.at[pl.ds(w0n, _CH), :],
                                 d_chunk2.at[ns], stsem)
                pltpu.async_copy(w2_hbm.at[pl.ds(w0n, _CH), :],
                                 w_chunk2.at[ns], stsem)

            descs = [
                pltpu.async_copy(w_chunk2.at[cs].at[j],
                                 deg_sh.at[d_chunk2.at[cs].at[j]],
                                 psem, add=True)
                for j in range(_CH)
            ]
            for dsc in descs:
                dsc.wait()

            @pl.when(cki < NCH1 - 1)
            def _():
                pltpu.make_async_copy(d2_hbm.at[pl.ds(w0n, _CH), :],
                                      d_chunk2.at[ns], stsem).wait()
                pltpu.make_async_copy(w2_hbm.at[pl.ds(w0n, _CH), :],
                                      w_chunk2.at[ns], stsem).wait()

        plsc.subcore_barrier()

        pltpu.sync_copy(deg_sh, isd_l)

        @pl.loop(0, NP // _L)
        def _(t):
            d = isd_l[pl.ds(t * _L, _L)] + 1.0
            i = plsc.bitcast(d, jnp.int32)
            y = plsc.bitcast(jnp.int32(0x5F3759DF) - (i >> 1), jnp.float32)
            y = y * (1.5 - 0.5 * d * y * y)
            y = y * (1.5 - 0.5 * d * y * y)
            y = y * (1.5 - 0.5 * d * y * y)
            isd_l[pl.ds(t * _L, _L)] = y

        @pl.when(c == 0)
        def _():
            pltpu.sync_copy(deg_sh.at[pl.ds(s * RPS, RPS)],
                            deg_hbm.at[pl.ds(s * RPS, RPS)])

        start3 = wid * W3

        def scale_window(cs, j, b):
            for kk in range(_WIN // _L):
                s16 = s_chunk2[cs, j, pl.ds(kk * _L, _L)]
                isd_s = plsc.load_gather(isd_l, [s16])
                cbuf[pl.ds(kk * _L, _L)] = (
                    w_chunk2[cs, j, pl.ds(kk * _L, _L)] * isd_s)

            @pl.loop(0, _WIN // _L)
            def _(g):
                c16 = cbuf[pl.ds(g * _L, _L)]
                for l in range(_L):
                    ce = c16[l]
                    e = g * _L + l
                    for jj in range(DG):
                        rows2[b, e, pl.ds(jj * _L, _L)] = (
                            rows2[b, e, pl.ds(jj * _L, _L)] * ce)

        pltpu.sync_copy(s2_hbm.at[pl.ds(start3, _CH), :], s_chunk2.at[0])
        pltpu.sync_copy(d2_hbm.at[pl.ds(start3, _CH), :], d_chunk2.at[0])
        pltpu.sync_copy(w2_hbm.at[pl.ds(start3, _CH), :], w_chunk2.at[0])
        pltpu.async_copy(x_hbm.at[s_chunk2.at[0].at[0]], rows2.at[0], gsem[0])

        @pl.loop(0, NCH3)
        def _(cki):
            cs = cki % 2
            ns = (cki + 1) % 2
            w0n = start3 + (cki + 1) * _CH
            st = []
            for j in range(_CH):
                b = j % 2
                nb = (j + 1) % 2
                if j == 0:
                    @pl.when(cki > 0)
                    def _():
                        pltpu.make_async_copy(
                            x_hbm.at[pl.ds(0, _WIN), :], rows2.at[nb],
                            tsem[nb]).wait()
                    pltpu.async_copy(x_hbm.at[s_chunk2.at[cs].at[j + 1]],
                                     rows2.at[nb], gsem[nb])

                    @pl.when(cki < NCH3 - 1)
                    def _():
                        st.append(pltpu.async_copy(
                            s2_hbm.at[pl.ds(w0n, _CH), :], s_chunk2.at[ns],
                            stsem))
                        st.append(pltpu.async_copy(
                            d2_hbm.at[pl.ds(w0n, _CH), :], d_chunk2.at[ns],
                            stsem))
                        st.append(pltpu.async_copy(
                            w2_hbm.at[pl.ds(w0n, _CH), :], w_chunk2.at[ns],
                            stsem))
                elif j < _CH - 1:
                    pltpu.make_async_copy(x_hbm.at[pl.ds(0, _WIN), :],
                                          rows2.at[nb], tsem[nb]).wait()
                    pltpu.async_copy(x_hbm.at[s_chunk2.at[cs].at[j + 1]],
                                     rows2.at[nb], gsem[nb])
                else:
                    @pl.when(cki < NCH3 - 1)
                    def _():
                        for dsc in st:
                            dsc.wait()
                        pltpu.make_async_copy(x_hbm.at[pl.ds(0, _WIN), :],
                                              rows2.at[nb], tsem[nb]).wait()
                        pltpu.async_copy(x_hbm.at[s_chunk2.at[ns].at[0]],
                                         rows2.at[nb], gsem[nb])
                pltpu.make_async_copy(x_hbm.at[pl.ds(0, _WIN), :],
                                      rows2.at[b], gsem[b]).wait()
                scale_window(cs, j, b)
                pltpu.async_copy(rows2.at[b], agg_sh.at[d_chunk2.at[cs].at[j]],
                                 tsem[b], add=True)

        pltpu.make_async_copy(x_hbm.at[pl.ds(0, _WIN), :], rows2.at[0],
                              tsem[0]).wait()
        pltpu.make_async_copy(x_hbm.at[pl.ds(0, _WIN), :], rows2.at[1],
                              tsem[1]).wait()

        plsc.subcore_barrier()

        for t in range(RPS // _WIN):
            sl = pl.ds(s * RPS + t * _WIN, _WIN)
            pltpu.sync_copy(agg_sh.at[sl, :], p_hbm.at[c].at[sl, :])

    return k(x, src2, dst2, w2)


def _tc_finish(P, x, deg2, W, b2):
    N, D = x.shape
    RB = 2000
    assert N % RB == 0

    def body(p0_r, p1_r, x_r, deg_r, w_r, b_r, o_r):
        deg = deg_r[...] + 1.0
        agg = lax.rsqrt(deg) * (p0_r[0] + p1_r[0]) + x_r[...] / deg
        y = jnp.dot(agg, w_r[...], preferred_element_type=jnp.float32) + b_r[...]
        o_r[...] = jnp.maximum(y, 0.0)

    return pl.pallas_call(
        body,
        grid=(N // RB,),
        in_specs=[
            pl.BlockSpec((1, RB, D), lambda i: (0, i, 0)),
            pl.BlockSpec((1, RB, D), lambda i: (1, i, 0)),
            pl.BlockSpec((RB, D), lambda i: (i, 0)),
            pl.BlockSpec((RB, 1), lambda i: (i, 0)),
            pl.BlockSpec((D, D), lambda i: (0, 0)),
            pl.BlockSpec((1, D), lambda i: (0, 0)),
        ],
        out_specs=pl.BlockSpec((RB, D), lambda i: (i, 0)),
        out_shape=jax.ShapeDtypeStruct((N, D), jnp.float32),
    )(P, P, x, deg2, W, b2)


def kernel(x, edge_index, edge_weights, W, b):
    N, D = x.shape
    E = edge_index.shape[1]
    NP = 10240
    unit = _WIN * _NC * _NS * _CH
    EP = -(-E // unit) * unit
    pad = EP - E
    pad_idx = jnp.arange(pad, dtype=jnp.int32) % jnp.int32(N)
    src2 = jnp.concatenate([edge_index[0], pad_idx]).reshape(EP // _WIN, _WIN)
    dst2 = jnp.concatenate([edge_index[1], pad_idx]).reshape(EP // _WIN, _WIN)
    w2 = jnp.concatenate(
        [edge_weights, jnp.zeros((pad,), jnp.float32)]).reshape(EP // _WIN, _WIN)
    P, deg_raw = _sc_aggregate(x, src2, dst2, w2, NP)
    deg2 = deg_raw[:N].reshape(N, 1)
    b2 = b.reshape(1, D)
    return _tc_finish(P, x, deg2, W, b2)
